# Initial kernel scaffold; baseline (speedup 1.0000x reference)
import jax
import jax.numpy as jnp
from jax.experimental import pallas as pl


def kernel(node_ids, neighbor_lists, theta):
    raise NotImplementedError("write your pallas kernel here")


# speedup vs baseline: 1.0351x; 1.0351x over previous
import functools
import jax
import jax.numpy as jnp
from jax import lax
from jax.experimental import pallas as pl
from jax.experimental.pallas import tpu as pltpu
from jax.experimental.pallas import tpu_sc as plsc

NUM_NODES = 100000
EMBED_DIM = 128
BATCH = 10000
NUM_NEIGHBORS = 32

_NC, _NS = 2, 16
_NW = _NC * _NS
_B_PAD = 10240
_B_PER_W = _B_PAD // _NW
_CHUNK_IDX = 128
_B_PER_CHUNK = _CHUNK_IDX // NUM_NEIGHBORS
_CHUNKS_PER_W = _B_PER_W // _B_PER_CHUNK


def _tec_body(theta_hbm, idx_hbm, out_hbm, idx_v, rows_v, out_v, sem):
    wid = lax.axis_index("s") * _NC + lax.axis_index("c")
    pltpu.sync_copy(idx_hbm.at[pl.ds(wid * _CHUNKS_PER_W, _CHUNKS_PER_W)], idx_v)

    def chunk(j, _):
        pltpu.async_copy(theta_hbm.at[idx_v.at[j]], rows_v, sem).wait()
        for bb in range(_B_PER_CHUNK):
            ob = j * _B_PER_CHUNK + bb
            for d in range(EMBED_DIM // 16):
                sl = pl.ds(d * 16, 16)
                vals = [rows_v[bb * NUM_NEIGHBORS + k, sl]
                        for k in range(NUM_NEIGHBORS)]
                while len(vals) > 1:
                    vals = [vals[i] + vals[i + 1] for i in range(0, len(vals), 2)]
                out_v[ob, sl] = vals[0] * (1.0 / NUM_NEIGHBORS)
        return ()

    lax.fori_loop(0, _CHUNKS_PER_W, chunk, (), unroll=False)
    pltpu.sync_copy(out_v, out_hbm.at[pl.ds(wid * _B_PER_W, _B_PER_W)])


@jax.jit
def kernel(node_ids, neighbor_lists, theta):
    del node_ids
    nbr = jnp.zeros((_B_PAD, NUM_NEIGHBORS), jnp.int32)
    nbr = nbr.at[:BATCH].set(neighbor_lists)
    idx = nbr.reshape(_B_PAD * NUM_NEIGHBORS // _CHUNK_IDX, _CHUNK_IDX)

    mesh = plsc.VectorSubcoreMesh(core_axis_name="c", subcore_axis_name="s")
    out = pl.kernel(
        _tec_body,
        out_type=jax.ShapeDtypeStruct((_B_PAD, EMBED_DIM), jnp.float32),
        mesh=mesh,
        scratch_types=[
            pltpu.VMEM((_CHUNKS_PER_W, _CHUNK_IDX), jnp.int32),
            pltpu.VMEM((_CHUNK_IDX, EMBED_DIM), jnp.float32),
            pltpu.VMEM((_B_PER_W, EMBED_DIM), jnp.float32),
            pltpu.SemaphoreType.DMA,
        ],
    )(theta, idx)
    return out[:BATCH]


# speedup vs baseline: 1.2674x; 1.2245x over previous
import functools
import jax
import jax.numpy as jnp
from jax import lax
from jax.experimental import pallas as pl
from jax.experimental.pallas import tpu as pltpu
from jax.experimental.pallas import tpu_sc as plsc

NUM_NODES = 100000
EMBED_DIM = 128
BATCH = 10000
NUM_NEIGHBORS = 32

_NC, _NS = 2, 16
_NW = _NC * _NS
_B_PAD = 10240
_B_PER_W = _B_PAD // _NW
_CHUNK_IDX = 128
_B_PER_CHUNK = _CHUNK_IDX // NUM_NEIGHBORS
_CHUNKS_PER_W = _B_PER_W // _B_PER_CHUNK


_NBUF = 2


def _tec_body(theta_hbm, idx_hbm, out_hbm, idx_v, rows0, rows1, out_v,
              sem0, sem1):
    wid = lax.axis_index("s") * _NC + lax.axis_index("c")
    pltpu.sync_copy(idx_hbm.at[pl.ds(wid * _CHUNKS_PER_W, _CHUNKS_PER_W)], idx_v)
    bufs = (rows0, rows1)
    sems = (sem0, sem1)

    def start(c, b):
        pltpu.async_copy(theta_hbm.at[idx_v.at[c]], bufs[b], sems[b])

    def reduce(c, b):
        rows = bufs[b]
        for bb in range(_B_PER_CHUNK):
            ob = c * _B_PER_CHUNK + bb
            for d in range(EMBED_DIM // 16):
                sl = pl.ds(d * 16, 16)
                vals = [rows[bb * NUM_NEIGHBORS + k, sl]
                        for k in range(NUM_NEIGHBORS)]
                while len(vals) > 1:
                    vals = [vals[i] + vals[i + 1] for i in range(0, len(vals), 2)]
                out_v[ob, sl] = vals[0] * (1.0 / NUM_NEIGHBORS)

    for b in range(_NBUF):
        start(b, b)

    def step(j, _):
        for b in range(_NBUF):
            c = j * _NBUF + b
            pltpu.make_async_copy(theta_hbm.at[idx_v.at[c]], bufs[b],
                                  sems[b]).wait()
            reduce(c, b)

            @pl.when(c + _NBUF < _CHUNKS_PER_W)
            def _():
                start(c + _NBUF, b)
        return ()

    lax.fori_loop(0, _CHUNKS_PER_W // _NBUF, step, (), unroll=False)
    pltpu.sync_copy(out_v, out_hbm.at[pl.ds(wid * _B_PER_W, _B_PER_W)])


@jax.jit
def kernel(node_ids, neighbor_lists, theta):
    del node_ids
    nbr = jnp.zeros((_B_PAD, NUM_NEIGHBORS), jnp.int32)
    nbr = nbr.at[:BATCH].set(neighbor_lists)
    idx = nbr.reshape(_B_PAD * NUM_NEIGHBORS // _CHUNK_IDX, _CHUNK_IDX)

    mesh = plsc.VectorSubcoreMesh(core_axis_name="c", subcore_axis_name="s")
    out = pl.kernel(
        _tec_body,
        out_type=jax.ShapeDtypeStruct((_B_PAD, EMBED_DIM), jnp.float32),
        mesh=mesh,
        scratch_types=[
            pltpu.VMEM((_CHUNKS_PER_W, _CHUNK_IDX), jnp.int32),
            pltpu.VMEM((_CHUNK_IDX, EMBED_DIM), jnp.float32),
            pltpu.VMEM((_CHUNK_IDX, EMBED_DIM), jnp.float32),
            pltpu.VMEM((_B_PER_W, EMBED_DIM), jnp.float32),
            pltpu.SemaphoreType.DMA,
            pltpu.SemaphoreType.DMA,
        ],
    )(theta, idx)
    return out[:BATCH]
